# baseline (speedup 1.0000x reference)
import functools

import jax
import jax.numpy as jnp
from jax.experimental import pallas as pl

B, S, D, H, V = 8, 2048, 256, 8, 256
DH = D // H
FF = 4 * D
PATCH = 16
P = S // PATCH
BLK = 128
NBLK = S // BLK
ENT_W = 8

_PREC = jax.lax.Precision.HIGHEST


def _mm(a, b):
    return jnp.dot(a, b, precision=_PREC)


def _mm_t(a, b):
    return jax.lax.dot_general(a, b, (((1,), (1,)), ((), ())), precision=_PREC)


def _ln(x, g, b):
    m = jnp.mean(x, axis=-1, keepdims=True)
    v = jnp.mean((x - m) ** 2, axis=-1, keepdims=True)
    return (x - m) * jax.lax.rsqrt(v + 1e-5) * g + b


def _softmax(x):
    x = x - jnp.max(x, axis=-1, keepdims=True)
    e = jnp.exp(x)
    return e / jnp.sum(e, axis=-1, keepdims=True)


def _layer(x, ln1g, ln1b, Wqkv, Wo, ln2g, ln2b, W1, b1, W2, b2, blk):
    n = x.shape[0]
    xn = _ln(x, ln1g, ln1b)
    qkv = _mm(xn, Wqkv)
    q, k, v = qkv[:, :D], qkv[:, D:2 * D], qkv[:, 2 * D:]
    scale = 1.0 / (DH ** 0.5)
    blocks = []
    for nb in range(n // blk):
        r = nb * blk
        heads = []
        for h in range(H):
            c = h * DH
            qh = q[r:r + blk, c:c + DH]
            kh = k[r:r + blk, c:c + DH]
            vh = v[r:r + blk, c:c + DH]
            s = _mm_t(qh, kh) * scale
            heads.append(_mm(_softmax(s), vh))
        blocks.append(jnp.concatenate(heads, axis=1))
    o = jnp.concatenate(blocks, axis=0)
    x = x + _mm(o, Wo)
    xn2 = _ln(x, ln2g, ln2b)
    hid = jax.nn.gelu(_mm(xn2, W1) + b1)
    return x + _mm(hid, W2) + b2


def _shift(x, d, pad):
    if d == 0:
        return x
    n = x.shape[1]
    padv = jnp.full((1, abs(d)), pad, x.dtype)
    if d > 0:
        return jnp.concatenate([x[:, d:], padv], axis=1)
    return jnp.concatenate([padv, x[:, :n + d]], axis=1)


def _entropy_logit(bv, w):
    eq = {d: (bv == _shift(bv, d, -1)).astype(jnp.float32)
          for d in range(-(ENT_W - 1), ENT_W)}
    prod = None
    for j in range(ENT_W):
        f = eq[-j]
        for d in range(-j + 1, ENT_W - j):
            f = f + eq[d]
        c = _shift(f, j, 1.0)
        prod = c if prod is None else prod * c
    ent = 3.0 - (1.4426950408889634 / ENT_W) * jnp.log(prod)
    last = ent[:, S - ENT_W:S - ENT_W + 1]
    pos = jax.lax.broadcasted_iota(jnp.int32, (1, S), 1)
    ent = jnp.where(pos > S - ENT_W, last, ent)
    return -ent * w


def _fwd_kernel(bytes_ref, emb_ref,
                l_ln1g, l_ln1b, l_Wqkv, l_Wo, l_ln2g, l_ln2b,
                l_W1, l_b1, l_W2, l_b2,
                g_ln1g, g_ln1b, g_Wqkv, g_Wo, g_ln2g, g_ln2b,
                g_W1, g_b1, g_W2, g_b2,
                entw_ref, cWq, cWk, cWv, cWo, outW, outb,
                out_ref):
    bv = bytes_ref[0]

    vocab = jax.lax.broadcasted_iota(jnp.int32, (V, S), 0)
    ohT = (vocab == bv).astype(jnp.float32)
    x0 = jax.lax.dot_general(ohT, emb_ref[...], (((0,), (0,)), ((), ())),
                             precision=_PREC)

    x = _layer(x0, l_ln1g[...], l_ln1b[...], l_Wqkv[...], l_Wo[...],
               l_ln2g[...], l_ln2b[...], l_W1[...], l_b1[...],
               l_W2[...], l_b2[...], BLK)

    logit = _entropy_logit(bv, entw_ref[0, 0])
    e = jnp.exp(logit)
    prow = jax.lax.broadcasted_iota(jnp.int32, (P, S), 0)
    pcol = jax.lax.broadcasted_iota(jnp.int32, (P, S), 1) // PATCH
    me = jnp.where(prow == pcol, e, 0.0)
    z = jnp.sum(me, axis=1, keepdims=True)
    pooled = _mm(me, x) / z

    g = _layer(pooled, g_ln1g[...], g_ln1b[...], g_Wqkv[...], g_Wo[...],
               g_ln2g[...], g_ln2b[...], g_W1[...], g_b1[...],
               g_W2[...], g_b2[...], P)

    qx = _mm(x, cWq[...])
    kg = _mm(g, cWk[...])
    vg = _mm(g, cWv[...])
    scale = 1.0 / (DH ** 0.5)
    heads = []
    for h in range(H):
        c = h * DH
        s = _mm_t(qx[:, c:c + DH], kg[:, c:c + DH]) * scale
        heads.append(_mm(_softmax(s), vg[:, c:c + DH]))
    o = jnp.concatenate(heads, axis=1)
    y = x + _mm(o, cWo[...])

    out_ref[0] = _mm(y, outW[...]) + outb[...]


def _full(shape):
    return pl.BlockSpec(shape, lambda i: (0,) * len(shape))


@functools.partial(jax.jit, static_argnames=())
def kernel(params, byte_sequence):
    bytes3 = byte_sequence.astype(jnp.int32).reshape(B, 1, S)
    lp, gp = params['local'], params['glob']

    def row(x):
        return x.reshape(1, -1)

    ops = [
        bytes3, params['emb'],
        row(lp['ln1_g']), row(lp['ln1_b']), lp['Wqkv'], lp['Wo'],
        row(lp['ln2_g']), row(lp['ln2_b']), lp['W1'], row(lp['b1']),
        lp['W2'], row(lp['b2']),
        row(gp['ln1_g']), row(gp['ln1_b']), gp['Wqkv'], gp['Wo'],
        row(gp['ln2_g']), row(gp['ln2_b']), gp['W1'], row(gp['b1']),
        gp['W2'], row(gp['b2']),
        params['entropy_weighting'].reshape(1, 1),
        params['cross_Wq'], params['cross_Wk'],
        params['cross_Wv'], params['cross_Wo'],
        params['out_W'], row(params['out_b']),
    ]
    in_specs = [pl.BlockSpec((1, 1, S), lambda i: (i, 0, 0))]
    in_specs += [_full(o.shape) for o in ops[1:]]
    return pl.pallas_call(
        _fwd_kernel,
        grid=(B,),
        in_specs=in_specs,
        out_specs=pl.BlockSpec((1, S, V), lambda i: (i, 0, 0)),
        out_shape=jax.ShapeDtypeStruct((B, S, V), jnp.float32),
    )(*ops)


# speedup vs baseline: 2.5376x; 2.5376x over previous
import functools

import jax
import jax.numpy as jnp
from jax.experimental import pallas as pl

B, S, D, H, V = 8, 2048, 256, 8, 256
DH = D // H
FF = 4 * D
PATCH = 16
P = S // PATCH
BLK = 128
NBLK = S // BLK
ENT_W = 8

_PREC = jax.lax.Precision.DEFAULT


def _mm(a, b):
    return jnp.dot(a, b, precision=_PREC)


def _mm_t(a, b):
    return jax.lax.dot_general(a, b, (((1,), (1,)), ((), ())), precision=_PREC)


def _ln(x, g, b):
    m = jnp.mean(x, axis=-1, keepdims=True)
    v = jnp.mean((x - m) ** 2, axis=-1, keepdims=True)
    return (x - m) * jax.lax.rsqrt(v + 1e-5) * g + b


def _softmax(x):
    x = x - jnp.max(x, axis=-1, keepdims=True)
    e = jnp.exp(x)
    return e / jnp.sum(e, axis=-1, keepdims=True)


def _layer(x, ln1g, ln1b, Wqkv, Wo, ln2g, ln2b, W1, b1, W2, b2, blk):
    n = x.shape[0]
    xn = _ln(x, ln1g, ln1b)
    qkv = _mm(xn, Wqkv)
    q, k, v = qkv[:, :D], qkv[:, D:2 * D], qkv[:, 2 * D:]
    scale = 1.0 / (DH ** 0.5)
    blocks = []
    for nb in range(n // blk):
        r = nb * blk
        heads = []
        for h in range(H):
            c = h * DH
            qh = q[r:r + blk, c:c + DH]
            kh = k[r:r + blk, c:c + DH]
            vh = v[r:r + blk, c:c + DH]
            s = _mm_t(qh, kh) * scale
            heads.append(_mm(_softmax(s), vh))
        blocks.append(jnp.concatenate(heads, axis=1))
    o = jnp.concatenate(blocks, axis=0)
    x = x + _mm(o, Wo)
    xn2 = _ln(x, ln2g, ln2b)
    hid = jax.nn.gelu(_mm(xn2, W1) + b1)
    return x + _mm(hid, W2) + b2


def _shift(x, d, pad):
    if d == 0:
        return x
    n = x.shape[1]
    padv = jnp.full((1, abs(d)), pad, x.dtype)
    if d > 0:
        return jnp.concatenate([x[:, d:], padv], axis=1)
    return jnp.concatenate([padv, x[:, :n + d]], axis=1)


def _entropy_logit(bv, w):
    eq = {d: (bv == _shift(bv, d, -1)).astype(jnp.float32)
          for d in range(-(ENT_W - 1), ENT_W)}
    prod = None
    for j in range(ENT_W):
        f = eq[-j]
        for d in range(-j + 1, ENT_W - j):
            f = f + eq[d]
        c = _shift(f, j, 1.0)
        prod = c if prod is None else prod * c
    ent = 3.0 - (1.4426950408889634 / ENT_W) * jnp.log(prod)
    last = ent[:, S - ENT_W:S - ENT_W + 1]
    pos = jax.lax.broadcasted_iota(jnp.int32, (1, S), 1)
    ent = jnp.where(pos > S - ENT_W, last, ent)
    return -ent * w


def _fwd_kernel(bytes_ref, emb_ref,
                l_ln1g, l_ln1b, l_Wqkv, l_Wo, l_ln2g, l_ln2b,
                l_W1, l_b1, l_W2, l_b2,
                g_ln1g, g_ln1b, g_Wqkv, g_Wo, g_ln2g, g_ln2b,
                g_W1, g_b1, g_W2, g_b2,
                entw_ref, cWq, cWk, cWv, cWo, outW, outb,
                out_ref):
    bv = bytes_ref[0]

    vocab = jax.lax.broadcasted_iota(jnp.int32, (V, S), 0)
    ohT = (vocab == bv).astype(jnp.float32)
    x0 = jax.lax.dot_general(ohT, emb_ref[...], (((0,), (0,)), ((), ())),
                             precision=_PREC)

    x = _layer(x0, l_ln1g[...], l_ln1b[...], l_Wqkv[...], l_Wo[...],
               l_ln2g[...], l_ln2b[...], l_W1[...], l_b1[...],
               l_W2[...], l_b2[...], BLK)

    logit = _entropy_logit(bv, entw_ref[0, 0])
    e = jnp.exp(logit)
    prow = jax.lax.broadcasted_iota(jnp.int32, (P, S), 0)
    pcol = jax.lax.broadcasted_iota(jnp.int32, (P, S), 1) // PATCH
    me = jnp.where(prow == pcol, e, 0.0)
    z = jnp.sum(me, axis=1, keepdims=True)
    pooled = _mm(me, x) / z

    g = _layer(pooled, g_ln1g[...], g_ln1b[...], g_Wqkv[...], g_Wo[...],
               g_ln2g[...], g_ln2b[...], g_W1[...], g_b1[...],
               g_W2[...], g_b2[...], P)

    qx = _mm(x, cWq[...])
    kg = _mm(g, cWk[...])
    vg = _mm(g, cWv[...])
    scale = 1.0 / (DH ** 0.5)
    heads = []
    for h in range(H):
        c = h * DH
        s = _mm_t(qx[:, c:c + DH], kg[:, c:c + DH]) * scale
        heads.append(_mm(_softmax(s), vg[:, c:c + DH]))
    o = jnp.concatenate(heads, axis=1)
    y = x + _mm(o, cWo[...])

    out_ref[0] = _mm(y, outW[...]) + outb[...]


def _full(shape):
    return pl.BlockSpec(shape, lambda i: (0,) * len(shape))


@functools.partial(jax.jit, static_argnames=())
def kernel(params, byte_sequence):
    bytes3 = byte_sequence.astype(jnp.int32).reshape(B, 1, S)
    lp, gp = params['local'], params['glob']

    def row(x):
        return x.reshape(1, -1)

    ops = [
        bytes3, params['emb'],
        row(lp['ln1_g']), row(lp['ln1_b']), lp['Wqkv'], lp['Wo'],
        row(lp['ln2_g']), row(lp['ln2_b']), lp['W1'], row(lp['b1']),
        lp['W2'], row(lp['b2']),
        row(gp['ln1_g']), row(gp['ln1_b']), gp['Wqkv'], gp['Wo'],
        row(gp['ln2_g']), row(gp['ln2_b']), gp['W1'], row(gp['b1']),
        gp['W2'], row(gp['b2']),
        params['entropy_weighting'].reshape(1, 1),
        params['cross_Wq'], params['cross_Wk'],
        params['cross_Wv'], params['cross_Wo'],
        params['out_W'], row(params['out_b']),
    ]
    in_specs = [pl.BlockSpec((1, 1, S), lambda i: (i, 0, 0))]
    in_specs += [_full(o.shape) for o in ops[1:]]
    return pl.pallas_call(
        _fwd_kernel,
        grid=(B,),
        in_specs=in_specs,
        out_specs=pl.BlockSpec((1, S, V), lambda i: (i, 0, 0)),
        out_shape=jax.ShapeDtypeStruct((B, S, V), jnp.float32),
    )(*ops)
